# Initial kernel scaffold; baseline (speedup 1.0000x reference)
import jax
import jax.numpy as jnp
from jax.experimental import pallas as pl


def kernel(mean, sigma, init_pos, z_noise, u_noise):
    raise NotImplementedError("write your pallas kernel here")


# speedup vs baseline: 3.5175x; 3.5175x over previous
import math

import jax
import jax.numpy as jnp
from jax.experimental import pallas as pl
from jax.experimental.pallas import tpu as pltpu

SQRT_2PI = math.sqrt(2.0 * math.pi)
NWALKERS = 65536
NELEC = 4
NDIM = 3
NSTEP = 16
D = NELEC * NDIM
WR, WC = 512, 128
LR = 0.2
EPS = 1e-7


def _body(mean_ref, sigma_ref, z_ref, u_ref, out_ref, ms_s):
    k = pl.program_id(0)

    @pl.when(k == 0)
    def _init_params():
        ms_s[0:1, 0:NDIM] = mean_ref[...]
        ms_s[1:2, 0:NDIM] = sigma_ref[...]

    def mdim(i):
        return ms_s[0:1, i:i + 1]

    def sdim(i):
        return ms_s[1:2, i:i + 1]

    j3 = jax.lax.broadcasted_iota(jnp.int32, (D, 1, 1), 0) % NDIM
    mrow = jnp.where(j3 == 0, mdim(0), jnp.where(j3 == 1, mdim(1), mdim(2)))
    srow = jnp.where(j3 == 0, sdim(0), jnp.where(j3 == 1, sdim(1), sdim(2)))

    z = z_ref[0]
    x = mrow + srow * z

    @pl.when(k == 0)
    def _init_pos():
        out_ref[...] = x

    @pl.when(k > 0)
    def _step():
        p = jnp.exp(-(x * x) / 2.0) / SQRT_2PI
        f = p[0]
        for j in range(1, D):
            f = f * p[j]

        g = None
        for i in range(NDIM):
            t = x[i] - mdim(i)
            gt = jnp.exp(-(t * t) / (2.0 * sdim(i) * sdim(i))) / (
                SQRT_2PI * sdim(i))
            g = gt if g is None else g * gt

        mmax = jnp.max(f / g)
        u = u_ref[0]
        accepted = (g * u) * mmax < f
        out_ref[...] = jnp.where(accepted[None], x, out_ref[...])

        coef = (2.0 / NWALKERS) * (g - f) * g
        for i in range(NDIM):
            si = sdim(i)
            inv_s2 = 1.0 / (si * si)
            t = x[i] - mdim(i)
            gm = jnp.sum(coef * t * inv_s2, keepdims=True)
            gs = jnp.sum(coef * (t * t * inv_s2 / si - 1.0 / si),
                         keepdims=True)
            ms_s[0:1, i:i + 1] = mdim(i) - LR * gm
            ms_s[1:2, i:i + 1] = jnp.maximum(sdim(i) - LR * gs, EPS)


def kernel(mean, sigma, init_pos, z_noise, u_noise):
    del init_pos
    z_t = z_noise.reshape(NSTEP + 1, NWALKERS, D).transpose(0, 2, 1)
    z_t = z_t.reshape(NSTEP + 1, D, WR, WC)
    u_r = u_noise.reshape(NSTEP, WR, WC)
    mean_r = mean.reshape(1, NDIM)
    sigma_r = sigma.reshape(1, NDIM)

    out_t = pl.pallas_call(
        _body,
        grid=(NSTEP + 1,),
        in_specs=[
            pl.BlockSpec((1, NDIM), lambda k: (0, 0)),
            pl.BlockSpec((1, NDIM), lambda k: (0, 0)),
            pl.BlockSpec((1, D, WR, WC), lambda k: (k, 0, 0, 0)),
            pl.BlockSpec((1, WR, WC), lambda k: (jnp.maximum(k - 1, 0), 0, 0)),
        ],
        out_specs=pl.BlockSpec((D, WR, WC), lambda k: (0, 0, 0)),
        out_shape=jax.ShapeDtypeStruct((D, WR, WC), jnp.float32),
        scratch_shapes=[pltpu.VMEM((2, 128), jnp.float32)],
    )(mean_r, sigma_r, z_t, u_r)

    return out_t.reshape(D, NWALKERS).T.reshape(NWALKERS, D)
